# baseline (speedup 1.0000x reference)
import jax
import jax.numpy as jnp
from jax import lax
from jax.experimental import pallas as pl
from jax.experimental.pallas import tpu as pltpu
from jax.experimental.pallas import tpu_sc as plsc

_M, _D, _B = 100000, 128, 16384
_CW = 16
_NCG = _D // _CW
_NRP = 4
_RPR = _M // _NRP
_NCH = 4
_R = _RPR // _NCH
_S = 256
_NP = _B // _S
_NB = 3


def _body(inp_hbm, idx_hbm, src_hbm, out_hbm,
          acc, b0, b1, b2, c0b, c1b, c2b, s0, s1, s2, semw0, semw1):
  ibufs = (b0, b1, b2)
  sbufs = (c0b, c1b, c2b)
  sems = (s0, s1, s2)
  cid = lax.axis_index("c")
  sid = lax.axis_index("s")
  wid = sid * 2 + cid
  cg = wid % _NCG
  rp = wid // _NCG
  c0 = cg * _CW
  lanes = lax.iota(jnp.int32, 16)

  def start(piece, t):
    pltpu.async_copy(
        idx_hbm.at[pl.ds(piece * _S, _S), pl.ds(c0, _CW)], ibufs[t], sems[t])
    pltpu.async_copy(
        src_hbm.at[pl.ds(piece * _S, _S), pl.ds(c0, _CW)], sbufs[t], sems[t])

  def wait(piece, t):
    pltpu.make_async_copy(
        idx_hbm.at[pl.ds(piece * _S, _S), pl.ds(c0, _CW)],
        ibufs[t], sems[t]).wait()
    pltpu.make_async_copy(
        src_hbm.at[pl.ds(piece * _S, _S), pl.ds(c0, _CW)],
        sbufs[t], sems[t]).wait()

  def consume(r0, t):
    ib = ibufs[t]
    sb = sbufs[t]

    @plsc.parallel_loop(0, _S, unroll=32)
    def row(r):
      iv = ib[r]
      sv = sb[r]
      loc = iv - r0
      msk = plsc.bitcast(loc, jnp.uint32) < jnp.uint32(_R)
      plsc.addupdate_scatter(acc, [loc, lanes], sv, mask=msk)

  def chunk(ch, carry):
    r0 = rp * _RPR + ch * _R

    def triple(j, c2):
      p = 3 * j
      for t in range(_NB):
        wait(p + t, t)
        consume(r0, t)

        @pl.when(p + t + _NB < _NP)
        def _():
          start(p + t + _NB, t)
      return c2

    lax.fori_loop(0, (_NP - 1) // _NB, triple, 0)
    wait(_NP - 1, (_NP - 1) % _NB)
    consume(r0, (_NP - 1) % _NB)

    for t in range(_NB):
      start(t, t)

    h = _R // 2
    r1 = r0 + _R
    wa = pltpu.make_async_copy(
        acc.at[pl.ds(0, h), :], out_hbm.at[pl.ds(r0, h), pl.ds(c0, _CW)],
        semw0)
    wb = pltpu.make_async_copy(
        acc.at[pl.ds(h, h), :], out_hbm.at[pl.ds(r0 + h, h), pl.ds(c0, _CW)],
        semw1)
    wa.start()
    wb.start()
    wa.wait()

    @pl.when(ch + 1 < _NCH)
    def _():
      pltpu.sync_copy(inp_hbm.at[pl.ds(r1, h), pl.ds(c0, _CW)],
                      acc.at[pl.ds(0, h), :])
    wb.wait()

    @pl.when(ch + 1 < _NCH)
    def _():
      pltpu.sync_copy(inp_hbm.at[pl.ds(r1 + h, h), pl.ds(c0, _CW)],
                      acc.at[pl.ds(h, h), :])
    return carry

  for t in range(_NB):
    start(t, t)
  pltpu.sync_copy(inp_hbm.at[pl.ds(rp * _RPR, _R), pl.ds(c0, _CW)], acc)
  lax.fori_loop(0, _NCH, chunk, 0)
  for t in range(_NB):
    wait(t, t)


@jax.jit
def _scatter_add(inp, idx, src):
  mesh = plsc.VectorSubcoreMesh(core_axis_name="c", subcore_axis_name="s")
  run = pl.kernel(
      _body,
      out_type=jax.ShapeDtypeStruct((_M, _D), jnp.float32),
      mesh=mesh,
      compiler_params=pltpu.CompilerParams(use_tc_tiling_on_sc=False,
                                           needs_layout_passes=False),
      scratch_types=[
          pltpu.VMEM((_R, _CW), jnp.float32),
          pltpu.VMEM((_S, _CW), jnp.int32),
          pltpu.VMEM((_S, _CW), jnp.int32),
          pltpu.VMEM((_S, _CW), jnp.int32),
          pltpu.VMEM((_S, _CW), jnp.float32),
          pltpu.VMEM((_S, _CW), jnp.float32),
          pltpu.VMEM((_S, _CW), jnp.float32),
          pltpu.SemaphoreType.DMA,
          pltpu.SemaphoreType.DMA,
          pltpu.SemaphoreType.DMA,
          pltpu.SemaphoreType.DMA,
          pltpu.SemaphoreType.DMA,
      ],
  )
  return run(inp, idx, src)


def kernel(input, dim, index, src):
  idx = (index + dim).astype(jnp.int32)
  return _scatter_add(input, idx, src)


# speedup vs baseline: 1.0420x; 1.0420x over previous
import jax
import jax.numpy as jnp
from jax import lax
from jax.experimental import pallas as pl
from jax.experimental.pallas import tpu as pltpu
from jax.experimental.pallas import tpu_sc as plsc

_M, _D, _B = 100000, 128, 16384
_CW = 16
_NCG = _D // _CW
_NRP = 4
_RPR = _M // _NRP
_NCH = 4
_R = _RPR // _NCH
_S = 256
_NP = _B // _S
_NB = 3


def _body(inp_hbm, idx_hbm, src_hbm, out_hbm,
          acc, b0, b1, b2, c0b, c1b, c2b, s0, s1, s2, semw0, semw1):
  ibufs = (b0, b1, b2)
  sbufs = (c0b, c1b, c2b)
  sems = (s0, s1, s2)
  cid = lax.axis_index("c")
  sid = lax.axis_index("s")
  wid = sid * 2 + cid
  cg = wid % _NCG
  rp = wid // _NCG
  c0 = cg * _CW
  lanes = lax.iota(jnp.int32, 16)

  def start(piece, t):
    pltpu.async_copy(
        idx_hbm.at[pl.ds(piece * _S, _S), pl.ds(c0, _CW)], ibufs[t], sems[t])
    pltpu.async_copy(
        src_hbm.at[pl.ds(piece * _S, _S), pl.ds(c0, _CW)], sbufs[t], sems[t])

  def wait(piece, t):
    pltpu.make_async_copy(
        idx_hbm.at[pl.ds(piece * _S, _S), pl.ds(c0, _CW)],
        ibufs[t], sems[t]).wait()
    pltpu.make_async_copy(
        src_hbm.at[pl.ds(piece * _S, _S), pl.ds(c0, _CW)],
        sbufs[t], sems[t]).wait()

  def consume(r0, t):
    ib = ibufs[t]
    sb = sbufs[t]

    @plsc.parallel_loop(0, _S, unroll=16)
    def row(r):
      iv = ib[r]
      sv = sb[r]
      loc = iv - r0
      msk = plsc.bitcast(loc, jnp.uint32) < jnp.uint32(_R)
      plsc.addupdate_scatter(acc, [loc, lanes], sv, mask=msk)

  def chunk(ch, carry):
    r0 = rp * _RPR + ch * _R

    def triple(j, c2):
      p = 3 * j
      for t in range(_NB):
        wait(p + t, t)
        consume(r0, t)

        @pl.when(p + t + _NB < _NP)
        def _():
          start(p + t + _NB, t)
      return c2

    lax.fori_loop(0, (_NP - 1) // _NB, triple, 0)
    wait(_NP - 1, (_NP - 1) % _NB)
    consume(r0, (_NP - 1) % _NB)

    for t in range(_NB):
      start(t, t)

    h = _R // 2
    r1 = r0 + _R
    wa = pltpu.make_async_copy(
        acc.at[pl.ds(0, h), :], out_hbm.at[pl.ds(r0, h), pl.ds(c0, _CW)],
        semw0)
    wb = pltpu.make_async_copy(
        acc.at[pl.ds(h, h), :], out_hbm.at[pl.ds(r0 + h, h), pl.ds(c0, _CW)],
        semw1)
    wa.start()
    wb.start()
    wa.wait()

    @pl.when(ch + 1 < _NCH)
    def _():
      pltpu.sync_copy(inp_hbm.at[pl.ds(r1, h), pl.ds(c0, _CW)],
                      acc.at[pl.ds(0, h), :])
    wb.wait()

    @pl.when(ch + 1 < _NCH)
    def _():
      pltpu.sync_copy(inp_hbm.at[pl.ds(r1 + h, h), pl.ds(c0, _CW)],
                      acc.at[pl.ds(h, h), :])
    return carry

  for t in range(_NB):
    start(t, t)
  pltpu.sync_copy(inp_hbm.at[pl.ds(rp * _RPR, _R), pl.ds(c0, _CW)], acc)
  lax.fori_loop(0, _NCH, chunk, 0)
  for t in range(_NB):
    wait(t, t)


@jax.jit
def _scatter_add(inp, idx, src):
  mesh = plsc.VectorSubcoreMesh(core_axis_name="c", subcore_axis_name="s")
  run = pl.kernel(
      _body,
      out_type=jax.ShapeDtypeStruct((_M, _D), jnp.float32),
      mesh=mesh,
      compiler_params=pltpu.CompilerParams(use_tc_tiling_on_sc=False,
                                           needs_layout_passes=False),
      scratch_types=[
          pltpu.VMEM((_R, _CW), jnp.float32),
          pltpu.VMEM((_S, _CW), jnp.int32),
          pltpu.VMEM((_S, _CW), jnp.int32),
          pltpu.VMEM((_S, _CW), jnp.int32),
          pltpu.VMEM((_S, _CW), jnp.float32),
          pltpu.VMEM((_S, _CW), jnp.float32),
          pltpu.VMEM((_S, _CW), jnp.float32),
          pltpu.SemaphoreType.DMA,
          pltpu.SemaphoreType.DMA,
          pltpu.SemaphoreType.DMA,
          pltpu.SemaphoreType.DMA,
          pltpu.SemaphoreType.DMA,
      ],
  )
  return run(inp, idx, src)


def kernel(input, dim, index, src):
  idx = (index + dim).astype(jnp.int32)
  return _scatter_add(input, idx, src)
